# Initial kernel scaffold; baseline (speedup 1.0000x reference)
import jax
import jax.numpy as jnp
from jax.experimental import pallas as pl


def kernel(hidden_states, W):
    raise NotImplementedError("write your pallas kernel here")


# speedup vs baseline: 5.9140x; 5.9140x over previous
import functools

import jax
import jax.numpy as jnp
from jax.experimental import pallas as pl
from jax.experimental.pallas import tpu as pltpu

_TOPK = 8
_NEG = -3.0e38


def _router_body(x_ref, wt_ref, out_ref):
    logits = jax.lax.dot_general(
        x_ref[...], wt_ref[...],
        dimension_numbers=(((1,), (0,)), ((), ())),
        preferred_element_type=jnp.float32,
        precision=jax.lax.Precision.DEFAULT,
    )
    work = logits
    m0 = None
    z = None
    thr = None
    for _ in range(_TOPK):
        m = jnp.max(work, axis=1, keepdims=True)
        if m0 is None:
            m0 = m
            z = jnp.ones_like(m)
        else:
            z = z + jnp.exp(m - m0)
        thr = m
        work = jnp.where(work >= m, _NEG, work)
    gates = jnp.where(logits >= thr, jnp.exp(logits - m0), 0.0)
    out_ref[...] = gates / z


def kernel(hidden_states, W):
    n, d = hidden_states.shape
    e = W.shape[0]
    wt = W.T
    bm = 512
    return pl.pallas_call(
        _router_body,
        grid=(n // bm,),
        in_specs=[
            pl.BlockSpec((bm, d), lambda i: (i, 0)),
            pl.BlockSpec((d, e), lambda i: (0, 0)),
        ],
        out_specs=pl.BlockSpec((bm, e), lambda i: (i, 0)),
        out_shape=jax.ShapeDtypeStruct((n, e), jnp.float32),
        compiler_params=pltpu.CompilerParams(
            dimension_semantics=("parallel",),
        ),
    )(hidden_states, wt)
